# Initial kernel scaffold; baseline (speedup 1.0000x reference)
import jax
import jax.numpy as jnp
from jax.experimental import pallas as pl


def kernel(x, edge_index, W0, as0, ad0, b0, g0, bb0, W1, as1, ad1, b1, g1, bb1, W2, as2, ad2, b2):
    raise NotImplementedError("write your pallas kernel here")


# speedup vs baseline: 37.5838x; 37.5838x over previous
import functools

import numpy as np
import jax
import jax.numpy as jnp
from jax import lax
from jax.experimental import pallas as pl
from jax.experimental.pallas import tpu as pltpu
from jax.experimental.pallas import tpu_sc as plsc

N = 10000
E = 320000
D_IN = 128
HID = 16
HEADS = 8
NUM_LABEL = 64

NW = 32
C = 128
E_TOT = E + N
CPW = -(-E_TOT // (NW * C))
E_PAD = CPW * NW * C
N_PAD = 10240
ROWS_PT = N_PAD // 16
DUM = N


def _splat_i32(v):
    return jnp.full((16,), v, dtype=jnp.int32)


def _make_sc_edge(D, heads):
    ngrp = D // 16
    mesh = plsc.VectorSubcoreMesh(core_axis_name="c", subcore_axis_name="s")

    @functools.partial(
        pl.kernel,
        out_type=(
            jax.ShapeDtypeStruct((2, N_PAD, D), jnp.float32),
            jax.ShapeDtypeStruct((2, N_PAD, 16), jnp.float32),
        ),
        mesh=mesh,
        compiler_params=pltpu.CompilerParams(
            needs_layout_passes=False, use_tc_tiling_on_sc=False),
        scratch_types=[
            pltpu.VMEM((C,), jnp.int32),
            pltpu.VMEM((C,), jnp.int32),
            pltpu.VMEM((C, D), jnp.float32),
            pltpu.VMEM((C, 16), jnp.float32),
            pltpu.VMEM((C, 16), jnp.float32),
            pltpu.VMEM((C, 16), jnp.float32),
            pltpu.VMEM_SHARED((N_PAD, D), jnp.float32),
            pltpu.VMEM_SHARED((N_PAD, 16), jnp.float32),
            pltpu.SemaphoreType.DMA,
            pltpu.SemaphoreType.DMA,
            pltpu.SemaphoreType.DMA,
        ],
    )
    def sc_edge(h_hbm, comb_hbm, src_hbm, dst_hbm, num_hbm, den_hbm,
                src_v, dst_v, hs_v, cs_v, cd_v, ex_v, num_sp, den_sp,
                sem0, sem1, sem2):
        cid = lax.axis_index("c")
        sid = lax.axis_index("s")
        wid = cid * 16 + sid

        z16 = jnp.zeros((16,), jnp.float32)
        shift8 = lax.iota(jnp.int32, 16) % 8 + 8

        def zrow(r, _):
            for g in range(ngrp):
                hs_v[r, pl.ds(g * 16, 16)] = z16
            ex_v[r] = z16
            return 0
        lax.fori_loop(0, C, zrow, 0)
        for j in range(ROWS_PT // C):
            pltpu.sync_copy(hs_v, num_sp.at[pl.ds(sid * ROWS_PT + j * C, C)])
            pltpu.sync_copy(ex_v, den_sp.at[pl.ds(sid * ROWS_PT + j * C, C)])
        plsc.subcore_barrier()

        def chunk(k, _):
            ci = wid * CPW + k
            pltpu.sync_copy(src_hbm.at[ci], src_v)
            pltpu.sync_copy(dst_hbm.at[ci], dst_v)
            g1 = pltpu.async_copy(h_hbm.at[src_v], hs_v, sem0)
            g2 = pltpu.async_copy(comb_hbm.at[src_v], cs_v, sem1)
            g3 = pltpu.async_copy(comb_hbm.at[dst_v], cd_v, sem2)
            g1.wait()
            g2.wait()
            g3.wait()

            def edge(e, _):
                se = _splat_i32(e)
                csv = cs_v[e]
                cdr = plsc.load_gather(cd_v, [se, shift8])
                ev = csv + cdr
                ev = jnp.maximum(ev, 0.2 * ev)
                exv = jnp.exp(ev)
                ex_v[e] = exv
                for g in range(ngrp):
                    lane = g if heads == 8 else 0
                    b = plsc.load_gather(ex_v, [se, _splat_i32(lane)])
                    hs_v[e, pl.ds(g * 16, 16)] = hs_v[e, pl.ds(g * 16, 16)] * b
                return 0
            lax.fori_loop(0, C, edge, 0)

            s1 = pltpu.async_copy(hs_v, num_sp.at[dst_v], sem0, add=True)
            s2 = pltpu.async_copy(ex_v, den_sp.at[dst_v], sem1, add=True)
            s1.wait()
            s2.wait()
            return 0
        lax.fori_loop(0, CPW, chunk, 0)

        plsc.subcore_barrier()
        pltpu.sync_copy(num_sp.at[pl.ds(sid * ROWS_PT, ROWS_PT)],
                        num_hbm.at[cid, pl.ds(sid * ROWS_PT, ROWS_PT)])
        pltpu.sync_copy(den_sp.at[pl.ds(sid * ROWS_PT, ROWS_PT)],
                        den_hbm.at[cid, pl.ds(sid * ROWS_PT, ROWS_PT)])

    return sc_edge


_sc_edge_128 = _make_sc_edge(128, 8)
_sc_edge_64 = _make_sc_edge(64, 1)


def _rowmask(shape):
    return lax.broadcasted_iota(jnp.int32, shape, 0) < N


def _tc_first_body(x_ref, w_ref, asf_ref, adf_ref, s_ref, h_ref, comb_ref):
    h = jnp.dot(x_ref[...], w_ref[...], preferred_element_type=jnp.float32)
    asrc = jnp.dot(h * asf_ref[...], s_ref[...],
                   preferred_element_type=jnp.float32)
    adst = jnp.dot(h * adf_ref[...], s_ref[...],
                   preferred_element_type=jnp.float32)
    h_ref[...] = h
    comb_ref[...] = jnp.concatenate([asrc, adst], axis=1)


def _tc_mid_body(num_ref, den_ref, b_ref, g_ref, bb_ref, w_ref,
                 asf_ref, adf_ref, s_ref, r_ref, h_ref, comb_ref):
    nume = num_ref[0] + num_ref[1]
    dene = den_ref[0] + den_ref[1]
    denf = jnp.dot(dene, r_ref[...], preferred_element_type=jnp.float32)
    gat = nume / (denf + 1e-16) + b_ref[...]
    m = _rowmask(gat.shape)
    x = jnp.where(m, gat, 0.0)
    mu = jnp.sum(x, axis=0, keepdims=True) * (1.0 / N)
    xc = jnp.where(m, x - mu, 0.0)
    var = jnp.sum(xc * xc, axis=0, keepdims=True) * (1.0 / N)
    y = xc * lax.rsqrt(var + 1e-5) * g_ref[...] + bb_ref[...]
    act = jnp.where(y > 0, y, 0.2 * (jnp.exp(y) - 1.0))
    act = jnp.where(m, act, 0.0)
    h = jnp.dot(act, w_ref[...], preferred_element_type=jnp.float32)
    asrc = jnp.dot(h * asf_ref[...], s_ref[...],
                   preferred_element_type=jnp.float32)
    adst = jnp.dot(h * adf_ref[...], s_ref[...],
                   preferred_element_type=jnp.float32)
    h_ref[...] = h
    comb_ref[...] = jnp.concatenate([asrc, adst], axis=1)


def _tc_final_body(num_ref, den_ref, b_ref, r_ref, out_ref):
    nume = num_ref[0] + num_ref[1]
    dene = den_ref[0] + den_ref[1]
    denf = jnp.dot(dene, r_ref[...], preferred_element_type=jnp.float32)
    out_ref[...] = nume / (denf + 1e-16) + b_ref[...]


def _tc_first(x_pad, w, asf, adf, s):
    return pl.pallas_call(
        _tc_first_body,
        out_shape=(
            jax.ShapeDtypeStruct((N_PAD, w.shape[1]), jnp.float32),
            jax.ShapeDtypeStruct((N_PAD, 16), jnp.float32),
        ),
    )(x_pad, w, asf, adf, s)


def _tc_mid(num, den, b, g, bb, w, asf, adf, s, r):
    return pl.pallas_call(
        _tc_mid_body,
        out_shape=(
            jax.ShapeDtypeStruct((N_PAD, w.shape[1]), jnp.float32),
            jax.ShapeDtypeStruct((N_PAD, 16), jnp.float32),
        ),
    )(num, den, b, g, bb, w, asf, adf, s, r)


def _tc_final(num, den, b, r):
    return pl.pallas_call(
        _tc_final_body,
        out_shape=jax.ShapeDtypeStruct((N_PAD, NUM_LABEL), jnp.float32),
    )(num, den, b, r)


def kernel(x, edge_index, W0, as0, ad0, b0, g0, bb0,
           W1, as1, ad1, b1, g1, bb1, W2, as2, ad2, b2):
    loops = jnp.arange(N, dtype=edge_index.dtype)
    pad = jnp.full((E_PAD - E_TOT,), DUM, dtype=edge_index.dtype)
    src2d = jnp.concatenate([edge_index[0], loops, pad]).reshape(-1, C)
    dst2d = jnp.concatenate([edge_index[1], loops, pad]).reshape(-1, C)
    x_pad = jnp.zeros((N_PAD, D_IN), jnp.float32).at[:N].set(x)

    eye8 = jnp.eye(8, dtype=jnp.float32)
    s128 = jnp.repeat(eye8, 16, axis=0)
    s64 = jnp.zeros((64, 8), jnp.float32).at[:, 0].set(1.0)
    r128 = jnp.concatenate(
        [jnp.repeat(eye8, 16, axis=1), jnp.zeros((8, 128), jnp.float32)],
        axis=0)
    r64 = jnp.zeros((16, 64), jnp.float32).at[0].set(1.0)

    as0f, ad0f = as0.reshape(1, -1), ad0.reshape(1, -1)
    as1f, ad1f = as1.reshape(1, -1), ad1.reshape(1, -1)
    as2f, ad2f = as2.reshape(1, -1), ad2.reshape(1, -1)
    b0r, g0r, bb0r = b0.reshape(1, -1), g0.reshape(1, -1), bb0.reshape(1, -1)
    b1r, g1r, bb1r = b1.reshape(1, -1), g1.reshape(1, -1), bb1.reshape(1, -1)
    b2r = b2.reshape(1, -1)

    h0, comb0 = _tc_first(x_pad, W0, as0f, ad0f, s128)
    num0, den0 = _sc_edge_128(h0, comb0, src2d, dst2d)
    h1, comb1 = _tc_mid(num0, den0, b0r, g0r, bb0r, W1, as1f, ad1f, s128, r128)
    num1, den1 = _sc_edge_128(h1, comb1, src2d, dst2d)
    h2, comb2 = _tc_mid(num1, den1, b1r, g1r, bb1r, W2, as2f, ad2f, s64, r128)
    num2, den2 = _sc_edge_64(h2, comb2, src2d, dst2d)
    out = _tc_final(num2, den2, b2r, r64)
    return out[:N]


# speedup vs baseline: 55.7414x; 1.4831x over previous
import functools

import numpy as np
import jax
import jax.numpy as jnp
from jax import lax
from jax.experimental import pallas as pl
from jax.experimental.pallas import tpu as pltpu
from jax.experimental.pallas import tpu_sc as plsc

N = 10000
E = 320000
D_IN = 128
HID = 16
HEADS = 8
NUM_LABEL = 64

NW = 32
C = 72
E_TOT = E + N
CPW2 = -(-E_TOT // (NW * C))
E_PAD = CPW2 * NW * C
N_PAD = 10240
ROWS_PT = N_PAD // 16
DUM = N
assert CPW2 % 3 == 0


def _splat_i32(v):
    return jnp.full((16,), v, dtype=jnp.int32)


def _make_sc_edge(D, heads):
    ngrp = D // 16
    DW = D + 16
    mesh = plsc.VectorSubcoreMesh(core_axis_name="c", subcore_axis_name="s")

    @functools.partial(
        pl.kernel,
        out_type=jax.ShapeDtypeStruct((2, N_PAD, DW), jnp.float32),
        mesh=mesh,
        compiler_params=pltpu.CompilerParams(
            needs_layout_passes=False, use_tc_tiling_on_sc=False),
        scratch_types=[
            pltpu.VMEM((4, C), jnp.int32),
            pltpu.VMEM((4, C), jnp.int32),
            pltpu.VMEM((C, DW), jnp.float32),
            pltpu.VMEM((C, DW), jnp.float32),
            pltpu.VMEM((C, DW), jnp.float32),
            pltpu.VMEM((C, 16), jnp.float32),
            pltpu.VMEM((C, 16), jnp.float32),
            pltpu.VMEM((C, 16), jnp.float32),
            pltpu.VMEM_SHARED((N_PAD, DW), jnp.float32),
            pltpu.SemaphoreType.DMA,
            pltpu.SemaphoreType.DMA,
            pltpu.SemaphoreType.DMA,
            pltpu.SemaphoreType.DMA,
            pltpu.SemaphoreType.DMA,
            pltpu.SemaphoreType.DMA,
            pltpu.SemaphoreType.DMA,
        ],
    )
    def sc_edge(h_hbm, comb_hbm, src_hbm, dst_hbm, acc_hbm,
                idxs_v, idxd_v, hs0, hs1, hs2, cd0, cd1, cd2,
                acc_sp, semg0, semg1, semg2, sems0, sems1, sems2, semi):
        cid = lax.axis_index("c")
        sid = lax.axis_index("s")
        wid = cid * 16 + sid
        hs = (hs0, hs1, hs2)
        cd = (cd0, cd1, cd2)
        semg = (semg0, semg1, semg2)
        sems = (sems0, sems1, sems2)
        LAST = CPW2 - 1

        z16 = jnp.zeros((16,), jnp.float32)
        shift8 = lax.iota(jnp.int32, 16) % 8 + 8

        def zrow(r, _):
            for g in range(ngrp + 1):
                hs0[r, pl.ds(g * 16, 16)] = z16
            return 0
        lax.fori_loop(0, C, zrow, 0)
        base = sid * ROWS_PT
        nfull = ROWS_PT // C
        for j in range(nfull):
            pltpu.sync_copy(hs0, acc_sp.at[pl.ds(base + j * C, C)])
        rem = ROWS_PT - nfull * C
        if rem:
            pltpu.sync_copy(hs0.at[pl.ds(0, rem)],
                            acc_sp.at[pl.ds(base + nfull * C, rem)])
        plsc.subcore_barrier()

        def idx_copies(k):
            ksrc = jnp.minimum(k, LAST)
            r = k % 4
            return (
                pltpu.make_async_copy(src_hbm.at[wid, ksrc], idxs_v.at[r],
                                      semi),
                pltpu.make_async_copy(dst_hbm.at[wid, ksrc], idxd_v.at[r],
                                      semi),
            )

        def gathers(k, b):
            r = k % 4
            return (
                pltpu.make_async_copy(h_hbm.at[idxs_v.at[r]], hs[b], semg[b]),
                pltpu.make_async_copy(comb_hbm.at[idxd_v.at[r]], cd[b],
                                      semg[b]),
            )

        def scatter(k, b):
            return pltpu.make_async_copy(hs[b], acc_sp.at[idxd_v.at[k % 4]],
                                         sems[b])

        def compute(b):
            hsb, cdb = hs[b], cd[b]

            def edge(e, _):
                se = _splat_i32(e)
                asv = hsb[e, pl.ds(D, 16)]
                cdr = plsc.load_gather(cdb, [se, shift8])
                ev = asv + cdr
                ev = jnp.maximum(ev, 0.2 * ev)
                hsb[e, pl.ds(D, 16)] = jnp.exp(ev)
                for g in range(ngrp):
                    lane = D + (g if heads == 8 else 0)
                    w = plsc.load_gather(hsb, [se, _splat_i32(lane)])
                    hsb[e, pl.ds(g * 16, 16)] = hsb[e, pl.ds(g * 16, 16)] * w
                return 0
            lax.fori_loop(0, C, edge, 0)

        pre = idx_copies(0) + idx_copies(1)
        for c in pre:
            c.start()
        for c in pre:
            c.wait()
        for c in gathers(0, 0):
            c.start()

        def body(i, _):
            for j in range(3):
                k = 3 * i + j
                b = j
                bn = (j + 1) % 3
                if j < 2:
                    @pl.when(i > 0)
                    def _wait_prev(k=k, bn=bn):
                        scatter(k - 2, bn).wait()
                else:
                    scatter(k - 2, bn).wait()
                for c in idx_copies(k + 2):
                    c.start()
                for c in gathers(k + 1, bn):
                    c.start()
                for c in gathers(k, b):
                    c.wait()
                compute(b)
                for c in idx_copies(k + 2):
                    c.wait()
                scatter(k, b).start(add=True)
            return 0
        lax.fori_loop(0, CPW2 // 3, body, 0)

        for c in gathers(CPW2, 0):
            c.wait()
        scatter(CPW2 - 2, 1).wait()
        scatter(CPW2 - 1, 2).wait()

        plsc.subcore_barrier()
        pltpu.sync_copy(acc_sp.at[pl.ds(sid * ROWS_PT, ROWS_PT)],
                        acc_hbm.at[cid, pl.ds(sid * ROWS_PT, ROWS_PT)])

    return sc_edge


_sc_edge_128 = _make_sc_edge(128, 8)
_sc_edge_64 = _make_sc_edge(64, 1)


def _rowmask(shape):
    return lax.broadcasted_iota(jnp.int32, shape, 0) < N


def _attn_outputs(h, asf_ref, adf_ref, s_ref, h_ref, comb_ref):
    asrc = jnp.dot(h * asf_ref[...], s_ref[...],
                   preferred_element_type=jnp.float32)
    adst = jnp.dot(h * adf_ref[...], s_ref[...],
                   preferred_element_type=jnp.float32)
    zer8 = jnp.zeros((N_PAD, 8), jnp.float32)
    h_ref[...] = jnp.concatenate([h, asrc, zer8], axis=1)
    comb_ref[...] = jnp.concatenate([zer8, adst], axis=1)


def _tc_first_body(x_ref, w_ref, asf_ref, adf_ref, s_ref, h_ref, comb_ref):
    h = jnp.dot(x_ref[...], w_ref[...], preferred_element_type=jnp.float32)
    _attn_outputs(h, asf_ref, adf_ref, s_ref, h_ref, comb_ref)


def _tc_mid_body(acc_ref, b_ref, g_ref, bb_ref, w_ref,
                 asf_ref, adf_ref, s_ref, r_ref, h_ref, comb_ref):
    d_in = acc_ref.shape[2] - 16
    nume = acc_ref[0, :, :d_in] + acc_ref[1, :, :d_in]
    dene = acc_ref[0, :, d_in:] + acc_ref[1, :, d_in:]
    denf = jnp.dot(dene, r_ref[...], preferred_element_type=jnp.float32)
    gat = nume / (denf + 1e-16) + b_ref[...]
    m = _rowmask(gat.shape)
    x = jnp.where(m, gat, 0.0)
    mu = jnp.sum(x, axis=0, keepdims=True) * (1.0 / N)
    xc = jnp.where(m, x - mu, 0.0)
    var = jnp.sum(xc * xc, axis=0, keepdims=True) * (1.0 / N)
    y = xc * lax.rsqrt(var + 1e-5) * g_ref[...] + bb_ref[...]
    act = jnp.where(y > 0, y, 0.2 * (jnp.exp(y) - 1.0))
    act = jnp.where(m, act, 0.0)
    h = jnp.dot(act, w_ref[...], preferred_element_type=jnp.float32)
    _attn_outputs(h, asf_ref, adf_ref, s_ref, h_ref, comb_ref)


def _tc_final_body(acc_ref, b_ref, r_ref, out_ref):
    d_in = acc_ref.shape[2] - 16
    nume = acc_ref[0, :, :d_in] + acc_ref[1, :, :d_in]
    dene = acc_ref[0, :, d_in:] + acc_ref[1, :, d_in:]
    denf = jnp.dot(dene, r_ref[...], preferred_element_type=jnp.float32)
    out_ref[...] = nume / (denf + 1e-16) + b_ref[...]


def _tc_first(x_pad, w, asf, adf, s):
    return pl.pallas_call(
        _tc_first_body,
        out_shape=(
            jax.ShapeDtypeStruct((N_PAD, w.shape[1] + 16), jnp.float32),
            jax.ShapeDtypeStruct((N_PAD, 16), jnp.float32),
        ),
    )(x_pad, w, asf, adf, s)


def _tc_mid(acc, b, g, bb, w, asf, adf, s, r):
    return pl.pallas_call(
        _tc_mid_body,
        out_shape=(
            jax.ShapeDtypeStruct((N_PAD, w.shape[1] + 16), jnp.float32),
            jax.ShapeDtypeStruct((N_PAD, 16), jnp.float32),
        ),
    )(acc, b, g, bb, w, asf, adf, s, r)


def _tc_final(acc, b, r):
    return pl.pallas_call(
        _tc_final_body,
        out_shape=jax.ShapeDtypeStruct((N_PAD, NUM_LABEL), jnp.float32),
    )(acc, b, r)


def kernel(x, edge_index, W0, as0, ad0, b0, g0, bb0,
           W1, as1, ad1, b1, g1, bb1, W2, as2, ad2, b2):
    loops = jnp.arange(N, dtype=edge_index.dtype)
    pad = jnp.full((E_PAD - E_TOT,), DUM, dtype=edge_index.dtype)
    src2d = jnp.concatenate([edge_index[0], loops, pad]).reshape(NW, CPW2, C)
    dst2d = jnp.concatenate([edge_index[1], loops, pad]).reshape(NW, CPW2, C)
    x_pad = jnp.zeros((N_PAD, D_IN), jnp.float32).at[:N].set(x)

    eye8 = jnp.eye(8, dtype=jnp.float32)
    s128 = jnp.repeat(eye8, 16, axis=0)
    s64 = jnp.zeros((64, 8), jnp.float32).at[:, 0].set(1.0)
    r128 = jnp.concatenate(
        [jnp.repeat(eye8, 16, axis=1), jnp.zeros((8, 128), jnp.float32)],
        axis=0)
    r64 = jnp.zeros((16, 64), jnp.float32).at[0].set(1.0)

    as0f, ad0f = as0.reshape(1, -1), ad0.reshape(1, -1)
    as1f, ad1f = as1.reshape(1, -1), ad1.reshape(1, -1)
    as2f, ad2f = as2.reshape(1, -1), ad2.reshape(1, -1)
    b0r, g0r, bb0r = b0.reshape(1, -1), g0.reshape(1, -1), bb0.reshape(1, -1)
    b1r, g1r, bb1r = b1.reshape(1, -1), g1.reshape(1, -1), bb1.reshape(1, -1)
    b2r = b2.reshape(1, -1)

    h0, comb0 = _tc_first(x_pad, W0, as0f, ad0f, s128)
    acc0 = _sc_edge_128(h0, comb0, src2d, dst2d)
    h1, comb1 = _tc_mid(acc0, b0r, g0r, bb0r, W1, as1f, ad1f, s128, r128)
    acc1 = _sc_edge_128(h1, comb1, src2d, dst2d)
    h2, comb2 = _tc_mid(acc1, b1r, g1r, bb1r, W2, as2f, ad2f, s64, r128)
    acc2 = _sc_edge_64(h2, comb2, src2d, dst2d)
    out = _tc_final(acc2, b2r, r64)
    return out[:N]
